# baseline (speedup 1.0000x reference)
import functools

import jax
import jax.numpy as jnp
from jax import lax
from jax.experimental import pallas as pl
from jax.experimental.pallas import tpu as pltpu
from jax.experimental.pallas import tpu_sc as plsc

B = 16384
D = 128
NSEG = D // 16
NC = 2
NS = 16
NW = NC * NS
BPW = B // NW
CHUNKS = (64, 128, 128, 128, 64)
OFFS = tuple(sum(CHUNKS[:i]) for i in range(len(CHUNKS)))
CMAX = max(CHUNKS)
NCH = len(CHUNKS)
NBG = BPW // 128

_mesh = plsc.VectorSubcoreMesh(
    core_axis_name="c", subcore_axis_name="s", num_cores=NC, num_subcores=NS
)


@functools.partial(
    pl.kernel,
    out_type=jax.ShapeDtypeStruct((B,), jnp.float32),
    mesh=_mesh,
    scratch_types=[
        pltpu.VMEM((BPW,), jnp.int32),
        pltpu.VMEM((BPW,), jnp.int32),
        pltpu.VMEM((2, CMAX, D), jnp.float32),
        pltpu.VMEM((2, CMAX, D), jnp.float32),
        pltpu.VMEM((BPW,), jnp.float32),
        pltpu.VMEM((BPW,), jnp.float32),
        pltpu.VMEM((BPW,), jnp.float32),
        pltpu.VMEM((CMAX, 17), jnp.float32),
        pltpu.SemaphoreType.DMA,
        pltpu.SemaphoreType.DMA,
        pltpu.SemaphoreType.DMA,
        pltpu.SemaphoreType.DMA,
        pltpu.SemaphoreType.DMA,
    ],
    compiler_params=pltpu.CompilerParams(needs_layout_passes=False),
)
def _mf_sc(user_h, item_h, uemb_h, iemb_h, ubias_h, ibias_h, out_h,
           uidx, iidx, urows, vrows, ubv, ibv, outv, accb,
           sem_u0, sem_u1, sem_v0, sem_v1, sem_b):
    cid = lax.axis_index("c")
    sid = lax.axis_index("s")
    wid = sid * NC + cid
    pltpu.sync_copy(user_h.at[wid], uidx)
    pltpu.sync_copy(item_h.at[wid], iidx)

    sem_u = [sem_u0, sem_u1]
    sem_v = [sem_v0, sem_v1]

    def fire(ch):
        buf = ch % 2
        off, sz = OFFS[ch], CHUNKS[ch]
        isl = pl.ds(off, sz)
        dsl = pl.ds(0, sz)
        du = pltpu.async_copy(
            uemb_h.at[uidx.at[isl]], urows.at[buf, dsl], sem_u[buf])
        dv = pltpu.async_copy(
            iemb_h.at[iidx.at[isl]], vrows.at[buf, dsl], sem_v[buf])
        return du, dv

    pending = fire(0)
    bias_dmas = []
    for k in range(NBG):
        sl = pl.ds(k * 128, 128)
        bias_dmas.append(pltpu.async_copy(ubias_h.at[uidx.at[sl]],
                                          ubv.at[sl], sem_b))
        bias_dmas.append(pltpu.async_copy(ibias_h.at[iidx.at[sl]],
                                          ibv.at[sl], sem_b))

    for ch in range(NCH):
        pending[0].wait()
        pending[1].wait()
        buf = ch % 2
        if ch + 1 < NCH:
            pending = fire(ch + 1)
        ub = urows.at[buf]
        vb = vrows.at[buf]
        lane = lax.iota(jnp.int32, 16)
        off, sz = OFFS[ch], CHUNKS[ch]

        def row(r, carry, ub=ub, vb=vb):
            acc = ub[r, pl.ds(0, 16)] * vb[r, pl.ds(0, 16)]
            for s in range(1, NSEG):
                acc = acc + ub[r, pl.ds(s * 16, 16)] * vb[r, pl.ds(s * 16, 16)]
            accb[r, pl.ds(0, 16)] = acc
            return carry

        lax.fori_loop(0, sz, row, 0, unroll=4)

        if ch == 0:
            for d in bias_dmas:
                d.wait()

        def group(g, carry, off=off):
            rows16 = g * 16 + lane
            tot = plsc.load_gather(accb, [rows16, jnp.zeros((16,), jnp.int32)])
            for j in range(1, 16):
                tot = tot + plsc.load_gather(
                    accb, [rows16, jnp.full((16,), j, jnp.int32)])
            o = off + g * 16
            outv[pl.ds(o, 16)] = tot + ubv[pl.ds(o, 16)] + ibv[pl.ds(o, 16)]
            return carry

        lax.fori_loop(0, sz // 16, group, 0)

    pltpu.sync_copy(outv, out_h.at[pl.ds(wid * BPW, BPW)])


def kernel(user, item, user_emb, item_emb, user_bias, item_bias):
    u2 = user.reshape(NW, BPW)
    i2 = item.reshape(NW, BPW)
    return _mf_sc(u2, i2, user_emb, item_emb,
                  user_bias.reshape(-1), item_bias.reshape(-1))
